# Initial kernel scaffold; baseline (speedup 1.0000x reference)
import jax
import jax.numpy as jnp
from jax.experimental import pallas as pl


def kernel(x, edge_index, edge_type, batch, W_in, b_in, W_rel_0, root_0, bias_0, gamma_0, beta_0, W_rel_1, root_1, bias_1, gamma_1, beta_1, W_fin, b_fin):
    raise NotImplementedError("write your pallas kernel here")


# speedup vs baseline: 21.4370x; 21.4370x over previous
import functools

import jax
import jax.numpy as jnp
from jax import lax
from jax.experimental import pallas as pl
from jax.experimental.pallas import tpu as pltpu
from jax.experimental.pallas import tpu_sc as plsc

N = 10000
E = 320000
D = 128
R = 20
EPS = 1e-5
NR = N * R
KHALF = NR // 2
HROWS = 100160
CW = 16

NC = 2
NS = 16
L = 16

CH = 80

_mesh = plsc.VectorSubcoreMesh(core_axis_name="c", subcore_axis_name="s")
_sc_params = pltpu.CompilerParams(use_tc_tiling_on_sc=False,
                                  needs_layout_passes=False)


def _iota16():
    return lax.broadcasted_iota(jnp.int32, (L,), 0)


def _sc_counts(dst_hbm, typ_hbm, counts_hbm, hist, dstb, typb, keyb, onesb, zb):
    c = lax.axis_index("c")
    s = lax.axis_index("s")

    def zrow(i, _):
        zb[i] = jnp.zeros((L,), jnp.float32)
        return _
    lax.fori_loop(0, 626, zrow, None)
    for k in range(10):
        pltpu.sync_copy(zb, hist.at[pl.ds(s * 6260 + k * 626, 626)])

    e0 = jnp.where(_iota16() == 0, 1.0, 0.0).astype(jnp.float32)

    def orow(i, _):
        onesb[i] = e0
        return _
    lax.fori_loop(0, CH, orow, None)

    plsc.subcore_barrier()

    lo = c * KHALF
    epw = E // NS

    def chunk(j, _):
        base = pl.multiple_of(s * epw + j * CH, 8)
        pltpu.sync_copy(dst_hbm.at[pl.ds(base, CH)], dstb)
        pltpu.sync_copy(typ_hbm.at[pl.ds(base, CH)], typb)
        for g in range(CH // L):
            d16 = dstb[pl.ds(g * L, L)]
            t16 = typb[pl.ds(g * L, L)]
            key = d16 * R + t16 - lo
            ok = (key >= 0) & (key < KHALF)
            keyb[pl.ds(g * L, L)] = jnp.where(ok, key, KHALF)
        pltpu.sync_copy(onesb, hist.at[keyb], add=True)
        return _

    lax.fori_loop(0, epw // CH, chunk, None)
    plsc.subcore_barrier()

    rpt = KHALF // NS
    pltpu.sync_copy(hist.at[pl.ds(s * rpt, rpt)],
                    counts_hbm.at[pl.ds(c * KHALF + s * rpt, rpt)])


def _sc_weights(dst_hbm, typ_hbm, src_hbm, counts_hbm, w_hbm, gidx_hbm,
                dstb, typb, srcb, keyb, gixb, wb, rows, sem):
    c = lax.axis_index("c")
    s = lax.axis_index("s")
    wid = s * NC + c
    epw = E // (NC * NS)

    def chunk(j, _):
        base = pl.multiple_of(wid * epw + j * CH, 8)
        pltpu.sync_copy(dst_hbm.at[pl.ds(base, CH)], dstb)
        pltpu.sync_copy(typ_hbm.at[pl.ds(base, CH)], typb)
        pltpu.sync_copy(src_hbm.at[pl.ds(base, CH)], srcb)
        for g in range(CH // L):
            d16 = dstb[pl.ds(g * L, L)]
            t16 = typb[pl.ds(g * L, L)]
            s16 = srcb[pl.ds(g * L, L)]
            keyb[pl.ds(g * L, L)] = d16 * R + t16
            gixb[pl.ds(g * L, L)] = t16 * N + s16
        pltpu.async_copy(counts_hbm.at[keyb], rows, sem).wait()
        zero = jnp.zeros((L,), jnp.int32)
        for g in range(CH // L):
            c16 = plsc.load_gather(rows, [_iota16() + g * L, zero])
            wb[pl.ds(g * L, L)] = 1.0 / jnp.maximum(c16, 1.0)
        pltpu.sync_copy(wb, w_hbm.at[pl.ds(base, CH)])
        pltpu.sync_copy(gixb, gidx_hbm.at[pl.ds(base, CH)])
        return _

    lax.fori_loop(0, epw // CH, chunk, None)


def _sc_aggregate(h_hbm, gidx_hbm, w_hbm, dst_hbm, parts_hbm,
                  acc, gixb, wb, dstb, rows, zb, sem):
    c = lax.axis_index("c")
    s = lax.axis_index("s")
    rpt = N // NS

    def zrow(i, _):
        for k in range(D // L):
            zb[i, pl.ds(k * L, L)] = jnp.zeros((L,), jnp.float32)
        return _
    lax.fori_loop(0, 125, zrow, None)
    for k in range(rpt // 125):
        pltpu.sync_copy(zb, acc.at[pl.ds(s * rpt + k * 125, 125)])
    plsc.subcore_barrier()

    epw = E // (NC * NS)
    zero = jnp.zeros((L,), jnp.int32)

    def chunk(j, _):
        base = pl.multiple_of(c * (E // NC) + s * epw + j * CH, 8)
        pltpu.sync_copy(gidx_hbm.at[pl.ds(base, CH)], gixb)
        pltpu.sync_copy(w_hbm.at[pl.ds(base, CH)], wb)
        pltpu.sync_copy(dst_hbm.at[pl.ds(base, CH)], dstb)
        pltpu.async_copy(h_hbm.at[gixb], rows, sem).wait()

        def scale(i, _):
            wv = plsc.load_gather(wb, [zero + i])
            for k in range(D // L):
                rows[i, pl.ds(k * L, L)] = rows[i, pl.ds(k * L, L)] * wv
            return _
        lax.fori_loop(0, CH, scale, None)
        pltpu.sync_copy(rows, acc.at[dstb], add=True)
        return _

    lax.fori_loop(0, epw // CH, chunk, None)
    plsc.subcore_barrier()

    pltpu.sync_copy(acc.at[pl.ds(s * rpt, rpt)],
                    parts_hbm.at[c, pl.ds(s * rpt, rpt)])


_counts_call = pl.kernel(
    _sc_counts,
    out_type=jax.ShapeDtypeStruct((NR, CW), jnp.float32),
    mesh=_mesh,
    scratch_types=[
        pltpu.VMEM_SHARED((HROWS, CW), jnp.float32),
        pltpu.VMEM((CH,), jnp.int32),
        pltpu.VMEM((CH,), jnp.int32),
        pltpu.VMEM((CH,), jnp.int32),
        pltpu.VMEM((CH, CW), jnp.float32),
        pltpu.VMEM((626, CW), jnp.float32),
    ],
    compiler_params=_sc_params,
)

_weights_call = pl.kernel(
    _sc_weights,
    out_type=(jax.ShapeDtypeStruct((E,), jnp.float32),
              jax.ShapeDtypeStruct((E,), jnp.int32)),
    mesh=_mesh,
    scratch_types=[
        pltpu.VMEM((CH,), jnp.int32),
        pltpu.VMEM((CH,), jnp.int32),
        pltpu.VMEM((CH,), jnp.int32),
        pltpu.VMEM((CH,), jnp.int32),
        pltpu.VMEM((CH,), jnp.int32),
        pltpu.VMEM((CH,), jnp.float32),
        pltpu.VMEM((CH, CW), jnp.float32),
        pltpu.SemaphoreType.DMA,
    ],
    compiler_params=_sc_params,
)

_aggregate_call = pl.kernel(
    _sc_aggregate,
    out_type=jax.ShapeDtypeStruct((NC, N, D), jnp.float32),
    mesh=_mesh,
    scratch_types=[
        pltpu.VMEM_SHARED((N, D), jnp.float32),
        pltpu.VMEM((CH,), jnp.int32),
        pltpu.VMEM((CH,), jnp.float32),
        pltpu.VMEM((CH,), jnp.int32),
        pltpu.VMEM((CH, D), jnp.float32),
        pltpu.VMEM((125, D), jnp.float32),
        pltpu.SemaphoreType.DMA,
    ],
    compiler_params=_sc_params,
)


BN_ROWS = 1000


def _tc_in_body(x_ref, w_ref, b_ref, o_ref):
    o_ref[...] = jax.nn.relu(
        jnp.dot(x_ref[...], w_ref[...], preferred_element_type=jnp.float32)
        + b_ref[...])


def _tc_in(x, w, b):
    return pl.pallas_call(
        _tc_in_body,
        grid=(N // BN_ROWS,),
        in_specs=[
            pl.BlockSpec((BN_ROWS, D), lambda i: (i, 0)),
            pl.BlockSpec((D, D), lambda i: (0, 0)),
            pl.BlockSpec((1, D), lambda i: (0, 0)),
        ],
        out_specs=pl.BlockSpec((BN_ROWS, D), lambda i: (i, 0)),
        out_shape=jax.ShapeDtypeStruct((N, D), jnp.float32),
    )(x, w, b)


def _tc_rel_body(h_ref, w_ref, o_ref):
    o_ref[0] = jnp.dot(h_ref[...], w_ref[0], preferred_element_type=jnp.float32)


def _tc_rel(h, w_rel):
    return pl.pallas_call(
        _tc_rel_body,
        grid=(N // BN_ROWS, R),
        in_specs=[
            pl.BlockSpec((BN_ROWS, D), lambda i, r: (i, 0)),
            pl.BlockSpec((1, D, D), lambda i, r: (r, 0, 0)),
        ],
        out_specs=pl.BlockSpec((1, BN_ROWS, D), lambda i, r: (r, i, 0)),
        out_shape=jax.ShapeDtypeStruct((R, N, D), jnp.float32),
    )(h, w_rel)


def _tc_comb_body(fin, h_ref, root_ref, b_ref, p_ref, g_ref, be_ref,
                  wf_ref, bf_ref, o_ref):
    t = (jnp.dot(h_ref[...], root_ref[...], preferred_element_type=jnp.float32)
         + b_ref[...] + p_ref[0] + p_ref[1])
    t = t * (1.0 / jnp.sqrt(1.0 + EPS)) * g_ref[...] + be_ref[...]
    t = jax.nn.relu(t)
    if fin:
        o_ref[...] = jnp.sum(t * wf_ref[...], axis=1, keepdims=True) + bf_ref[...]
    else:
        o_ref[...] = t


def _tc_combine(h, root, bias, parts, gamma, beta, wf, bf, fin):
    out_shape = (N, 1) if fin else (N, D)
    out_block = (BN_ROWS, 1) if fin else (BN_ROWS, D)
    return pl.pallas_call(
        functools.partial(_tc_comb_body, fin),
        grid=(N // BN_ROWS,),
        in_specs=[
            pl.BlockSpec((BN_ROWS, D), lambda i: (i, 0)),
            pl.BlockSpec((D, D), lambda i: (0, 0)),
            pl.BlockSpec((1, D), lambda i: (0, 0)),
            pl.BlockSpec((NC, BN_ROWS, D), lambda i: (0, i, 0)),
            pl.BlockSpec((1, D), lambda i: (0, 0)),
            pl.BlockSpec((1, D), lambda i: (0, 0)),
            pl.BlockSpec((1, D), lambda i: (0, 0)),
            pl.BlockSpec((1, 1), lambda i: (0, 0)),
        ],
        out_specs=pl.BlockSpec(out_block, lambda i: (i, 0)),
        out_shape=jax.ShapeDtypeStruct(out_shape, jnp.float32),
    )(h, root, bias, parts, gamma, beta, wf, bf)


def kernel(x, edge_index, edge_type, batch, W_in, b_in,
           W_rel_0, root_0, bias_0, gamma_0, beta_0,
           W_rel_1, root_1, bias_1, gamma_1, beta_1, W_fin, b_fin):
    src = edge_index[0]
    dst = edge_index[1]

    counts = _counts_call(dst, edge_type)
    w_e, gidx = _weights_call(dst, edge_type, src, counts)

    b_in2 = b_in.reshape(1, D)
    bias_02 = bias_0.reshape(1, D)
    bias_12 = bias_1.reshape(1, D)
    gamma_02 = gamma_0.reshape(1, D)
    gamma_12 = gamma_1.reshape(1, D)
    beta_02 = beta_0.reshape(1, D)
    beta_12 = beta_1.reshape(1, D)
    wf2 = W_fin.reshape(1, D)
    bf2 = b_fin.reshape(1, 1)

    h0 = _tc_in(x, W_in, b_in2)

    H0 = _tc_rel(h0, W_rel_0).reshape(R * N, D)
    parts0 = _aggregate_call(H0, gidx, w_e, dst)
    h1 = _tc_combine(h0, root_0, bias_02, parts0, gamma_02, beta_02,
                     wf2, bf2, fin=False)

    H1 = _tc_rel(h1, W_rel_1).reshape(R * N, D)
    parts1 = _aggregate_call(H1, gidx, w_e, dst)
    out = _tc_combine(h1, root_1, bias_12, parts1, gamma_12, beta_12,
                      wf2, bf2, fin=True)
    return out
